# Initial kernel scaffold; baseline (speedup 1.0000x reference)
import jax
import jax.numpy as jnp
from jax.experimental import pallas as pl


def kernel(nodes, edges, senders, receivers, params):
    raise NotImplementedError("write your pallas kernel here")


# speedup vs baseline: 2.2787x; 2.2787x over previous
import functools

import jax
import jax.numpy as jnp
from jax import lax
from jax.experimental import pallas as pl
from jax.experimental.pallas import tpu as pltpu
from jax.experimental.pallas import tpu_sc as plsc

_N = 50000
_E = 1600000
_P = 8
_PN = 10
_EP = _E // _P
_NP = _N // _PN

_BE = 1000
_GE = _EP // _BE
_BN = 1000
_GN = _NP // _BN

_NC, _NS = 2, 16
_NW = _NC * _NS
_EW = _E // _NW
_CH = 2000
_NCH = _EW // _CH
_NPS = _N // _NS

_SELU_A = 1.6732632423543772
_SELU_S = 1.0507009873554805


def _selu(x):
    return _SELU_S * jnp.where(x > 0, x, _SELU_A * (jnp.exp(x) - 1.0))


def _dot(a, b):
    return jax.lax.dot_general(a, b, (((1,), (0,)), ((), ())))


def _bd(w, si, so, p=_P):
    fi, fo = w.shape
    wp = jnp.zeros((si, so), w.dtype).at[:fi, :fo].set(w)
    return jnp.kron(jnp.eye(p, dtype=w.dtype), wp)


def _bt(b, so, p=_P):
    bp = jnp.zeros((so,), b.dtype).at[: b.shape[0]].set(b)
    return jnp.tile(bp, p)[None, :]


def _edge_body(recv_ref, send_ref, edg_ref, eps_ref,
               w1r, w1s, w1e, b1, w2, b2, w3, b3, w34, b34,
               wd1, bd1, wd2, bd2,
               mu_ref, lv_ref, edec_ref, elat_ref, sum_ref):
    h = (_dot(recv_ref[...], w1r[...]) + _dot(send_ref[...], w1s[...])
         + _dot(edg_ref[...], w1e[...]) + b1[...])
    h = _selu(h)
    h = _selu(_dot(h, w2[...]) + b2[...])
    elat = _dot(h, w3[...]) + b3[...]
    elat_ref[...] = elat
    colsum = jnp.sum(elat, axis=0, keepdims=True)

    @pl.when(pl.program_id(0) == 0)
    def _():
        sum_ref[...] = colsum

    @pl.when(pl.program_id(0) != 0)
    def _():
        sum_ref[...] += colsum

    mulv = _dot(elat, w34[...]) + b34[...]
    mu = mulv[:, :160]
    lv = mulv[:, 160:]
    mu_ref[...] = mu
    lv_ref[...] = lv
    z = mu + eps_ref[...] * jnp.exp(0.5 * lv)
    d = _selu(_dot(z, wd1[...]) + bd1[...])
    edec_ref[...] = _dot(d, wd2[...]) + bd2[...]


def _edge_specs():
    c = lambda i: (0, 0)
    m = lambda i: (i, 0)
    in_specs = [
        pl.BlockSpec((_BE, 128), m),
        pl.BlockSpec((_BE, 128), m),
        pl.BlockSpec((_BE, 40), m),
        pl.BlockSpec((_BE, 160), m),
        pl.BlockSpec((128, 200), c), pl.BlockSpec((128, 200), c),
        pl.BlockSpec((40, 200), c), pl.BlockSpec((1, 200), c),
        pl.BlockSpec((200, 160), c), pl.BlockSpec((1, 160), c),
        pl.BlockSpec((160, 128), c), pl.BlockSpec((1, 128), c),
        pl.BlockSpec((128, 320), c), pl.BlockSpec((1, 320), c),
        pl.BlockSpec((160, 80), c), pl.BlockSpec((1, 80), c),
        pl.BlockSpec((80, 40), c), pl.BlockSpec((1, 40), c),
    ]
    out_specs = [
        pl.BlockSpec((_BE, 160), m),
        pl.BlockSpec((_BE, 160), m),
        pl.BlockSpec((_BE, 40), m),
        pl.BlockSpec((_BE, 128), m),
        pl.BlockSpec((1, 128), c),
    ]
    out_shape = [
        jax.ShapeDtypeStruct((_EP, 160), jnp.float32),
        jax.ShapeDtypeStruct((_EP, 160), jnp.float32),
        jax.ShapeDtypeStruct((_EP, 40), jnp.float32),
        jax.ShapeDtypeStruct((_EP, 128), jnp.float32),
        jax.ShapeDtypeStruct((1, 128), jnp.float32),
    ]
    return (_GE,), in_specs, out_specs, out_shape


def _node_body(ndp_ref, agg_ref, eps_ref, sume_ref,
               wnn, wna, bn1, wn2, bn2, wmn, bmn,
               wdn1, bdn1, wdn2, bdn2,
               wg0, bg0, wg1, bg1, wg2, bg2,
               mu_ref, lv_ref, ndec_ref, g_ref, sacc):
    agg = agg_ref[0] + agg_ref[1]
    h = _selu(_dot(ndp_ref[...], wnn[...]) + _dot(agg, wna[...]) + bn1[...])
    nlat = _dot(h, wn2[...]) + bn2[...]
    i = pl.program_id(0)
    colsum = jnp.sum(nlat, axis=0, keepdims=True)

    @pl.when(i == 0)
    def _():
        sacc[...] = colsum

    @pl.when(i != 0)
    def _():
        sacc[...] += colsum

    mulv = _dot(nlat, wmn[...]) + bmn[...]
    mu = mulv[:, :200]
    lv = mulv[:, 200:]
    mu_ref[...] = mu
    lv_ref[...] = lv
    z = mu + eps_ref[...] * jnp.exp(0.5 * lv)
    d = _selu(_dot(z, wdn1[...]) + bdn1[...])
    ndec_ref[...] = _dot(d, wdn2[...]) + bdn2[...]

    @pl.when(i == _GN - 1)
    def _():
        s_e = sume_ref[...]
        se10 = s_e[:, 0:10]
        for k in range(1, _P):
            se10 = se10 + s_e[:, 16 * k:16 * k + 10]
        s_n = sacc[...]
        sn10 = s_n[:, 0:10]
        for k in range(1, _PN):
            sn10 = sn10 + s_n[:, 10 * k:10 * k + 10]
        gi = jnp.concatenate([se10, sn10], axis=1)
        g = _selu(_dot(gi, wg0[...]) + bg0[...])
        g = _selu(_dot(g, wg1[...]) + bg1[...])
        g_ref[...] = _dot(g, wg2[...]) + bg2[...]


def _node_specs():
    c = lambda i: (0, 0)
    m = lambda i: (i, 0)
    in_specs = [
        pl.BlockSpec((_BN, 130), m),
        pl.BlockSpec((2, _BN, 160), lambda i: (0, i, 0)),
        pl.BlockSpec((_BN, 200), m),
        pl.BlockSpec((1, 128), c),
        pl.BlockSpec((130, 180), c), pl.BlockSpec((160, 180), c),
        pl.BlockSpec((1, 180), c),
        pl.BlockSpec((180, 100), c), pl.BlockSpec((1, 100), c),
        pl.BlockSpec((100, 400), c), pl.BlockSpec((1, 400), c),
        pl.BlockSpec((200, 160), c), pl.BlockSpec((1, 160), c),
        pl.BlockSpec((160, 130), c), pl.BlockSpec((1, 130), c),
        pl.BlockSpec((20, 15), c), pl.BlockSpec((1, 15), c),
        pl.BlockSpec((15, 15), c), pl.BlockSpec((1, 15), c),
        pl.BlockSpec((15, 10), c), pl.BlockSpec((1, 10), c),
    ]
    out_specs = [
        pl.BlockSpec((_BN, 200), m),
        pl.BlockSpec((_BN, 200), m),
        pl.BlockSpec((_BN, 130), m),
        pl.BlockSpec((1, 10), c),
    ]
    out_shape = [
        jax.ShapeDtypeStruct((_NP, 200), jnp.float32),
        jax.ShapeDtypeStruct((_NP, 200), jnp.float32),
        jax.ShapeDtypeStruct((_NP, 130), jnp.float32),
        jax.ShapeDtypeStruct((1, 10), jnp.float32),
    ]
    return (_GN,), in_specs, out_specs, out_shape


def _gather_call():
    mesh = plsc.VectorSubcoreMesh(core_axis_name="c", subcore_axis_name="s")

    @functools.partial(
        pl.kernel, mesh=mesh,
        compiler_params=pltpu.CompilerParams(use_tc_tiling_on_sc=False),
        out_type=(jax.ShapeDtypeStruct((_E, 16), jnp.float32),
                  jax.ShapeDtypeStruct((_E, 16), jnp.float32)),
        scratch_types=[
            pltpu.VMEM((_CH,), jnp.int32), pltpu.VMEM((_CH,), jnp.int32),
            pltpu.VMEM((_CH, 16), jnp.float32),
            pltpu.VMEM((_CH, 16), jnp.float32),
            pltpu.SemaphoreType.DMA, pltpu.SemaphoreType.DMA,
        ],
    )
    def gk(nodes_hbm, recv_hbm, send_hbm, outr_hbm, outs_hbm,
           idxr, idxs, rowr, rows, semr, sems):
        wid = lax.axis_index("s") * _NC + lax.axis_index("c")

        def body(i, carry):
            base = wid * _EW + i * _CH
            pltpu.sync_copy(recv_hbm.at[pl.ds(base, _CH)], idxr)
            cr = pltpu.async_copy(nodes_hbm.at[idxr], rowr, semr)
            pltpu.sync_copy(send_hbm.at[pl.ds(base, _CH)], idxs)
            cs = pltpu.async_copy(nodes_hbm.at[idxs], rows, sems)
            cr.wait()
            pltpu.sync_copy(rowr, outr_hbm.at[pl.ds(base, _CH)])
            cs.wait()
            pltpu.sync_copy(rows, outs_hbm.at[pl.ds(base, _CH)])
            return carry

        lax.fori_loop(0, _NCH, body, 0)

    return gk


def _scatter_call():
    mesh = plsc.VectorSubcoreMesh(core_axis_name="c", subcore_axis_name="s")

    @functools.partial(
        pl.kernel, mesh=mesh,
        compiler_params=pltpu.CompilerParams(use_tc_tiling_on_sc=False),
        out_type=jax.ShapeDtypeStruct((2 * _N, 16), jnp.float32),
        scratch_types=[
            pltpu.VMEM((_CH,), jnp.int32),
            pltpu.VMEM((_CH, 16), jnp.float32),
            pltpu.VMEM_SHARED((_N, 16), jnp.float32),
        ],
    )
    def sk(elat_hbm, recv_hbm, out_hbm, idx, rows, aggsh):
        cid = lax.axis_index("c")
        sid = lax.axis_index("s")
        zv = jnp.zeros((16,), jnp.float32)

        def zb(i, carry):
            rows[i, :] = zv
            return carry

        lax.fori_loop(0, _CH, zb, 0)
        pltpu.sync_copy(rows, aggsh.at[pl.ds(sid * _NPS, _CH)])
        pltpu.sync_copy(rows.at[pl.ds(0, _NPS - _CH)],
                        aggsh.at[pl.ds(sid * _NPS + _CH, _NPS - _CH)])
        plsc.subcore_barrier()

        def body(i, carry):
            base = (cid * _NS + sid) * _EW + i * _CH
            pltpu.sync_copy(recv_hbm.at[pl.ds(base, _CH)], idx)
            pltpu.sync_copy(elat_hbm.at[pl.ds(base, _CH)], rows)
            pltpu.sync_copy(rows, aggsh.at[idx], add=True)
            return carry

        lax.fori_loop(0, _NCH, body, 0)
        plsc.subcore_barrier()
        pltpu.sync_copy(aggsh.at[pl.ds(sid * _NPS, _NPS)],
                        out_hbm.at[pl.ds(cid * _N + sid * _NPS, _NPS)])

    return sk


def kernel(nodes, edges, senders, receivers, params):
    f32 = jnp.float32
    senders = senders.astype(jnp.int32)
    receivers = receivers.astype(jnp.int32)
    nodes16 = jnp.zeros((_N, 16), f32).at[:, :13].set(nodes)
    eps_n = jax.random.normal(jax.random.key(42), (_N, 20), dtype=f32)
    eps_e = jax.random.normal(jax.random.key(43), (_E, 20), dtype=f32)

    em = params["edge_mlp"]
    w1 = em[0]["W"]
    edge_w = [
        _bd(w1[5:18], 16, 25), _bd(w1[18:31], 16, 25), _bd(w1[:5], 5, 25),
        _bt(em[0]["b"], 25),
        _bd(em[1]["W"], 25, 20), _bt(em[1]["b"], 20),
        _bd(em[2]["W"], 20, 16), _bt(em[2]["b"], 16),
        jnp.concatenate([_bd(params["fc3"]["W"], 16, 20),
                         _bd(params["fc4"]["W"], 16, 20)], axis=1),
        jnp.concatenate([_bt(params["fc3"]["b"], 20),
                         _bt(params["fc4"]["b"], 20)], axis=1),
        _bd(params["dec_edge"][0]["W"], 20, 10), _bt(params["dec_edge"][0]["b"], 10),
        _bd(params["dec_edge"][1]["W"], 10, 5), _bt(params["dec_edge"][1]["b"], 5),
    ]

    nm = params["node_mlp"]
    wn1 = nm[0]["W"]
    node_w = [
        _bd(wn1[:13], 13, 18, _PN), _bd(wn1[13:23], 16, 18, _PN),
        _bt(nm[0]["b"], 18, _PN),
        _bd(nm[1]["W"], 18, 10, _PN), _bt(nm[1]["b"], 10, _PN),
        jnp.concatenate([_bd(params["fc1"]["W"], 10, 20, _PN),
                         _bd(params["fc2"]["W"], 10, 20, _PN)], axis=1),
        jnp.concatenate([_bt(params["fc1"]["b"], 20, _PN),
                         _bt(params["fc2"]["b"], 20, _PN)], axis=1),
        _bd(params["dec_node"][0]["W"], 20, 16, _PN),
        _bt(params["dec_node"][0]["b"], 16, _PN),
        _bd(params["dec_node"][1]["W"], 16, 13, _PN),
        _bt(params["dec_node"][1]["b"], 13, _PN),
    ]
    gm = params["global_mlp"]
    glob_w = [gm[0]["W"], gm[0]["b"][None, :],
              gm[1]["W"], gm[1]["b"][None, :],
              gm[2]["W"], gm[2]["b"][None, :]]

    recvf, sendf = _gather_call()(nodes16, receivers, senders)

    grid, in_specs, out_specs, out_shape = _edge_specs()
    mu_e_p, lv_e_p, edec_p, elat16_p, sum_e = pl.pallas_call(
        _edge_body, grid=grid, in_specs=in_specs, out_specs=out_specs,
        out_shape=out_shape,
    )(recvf.reshape(_EP, 128), sendf.reshape(_EP, 128),
      edges.reshape(_EP, 40), eps_e.reshape(_EP, 160), *edge_w)

    aggflat = _scatter_call()(elat16_p.reshape(_E, 16), receivers)

    grid, in_specs, out_specs, out_shape = _node_specs()
    mu_n_p, lv_n_p, ndec_p, g2 = pl.pallas_call(
        _node_body, grid=grid, in_specs=in_specs, out_specs=out_specs,
        out_shape=out_shape,
        scratch_shapes=[pltpu.VMEM((1, 100), f32)],
    )(nodes.reshape(_NP, 130), aggflat.reshape(2, _NP, 160),
      eps_n.reshape(_NP, 200), sum_e, *node_w, *glob_w)

    return (ndec_p.reshape(_N, 13), edec_p.reshape(_E, 5), g2.reshape(10),
            mu_n_p.reshape(_N, 20), lv_n_p.reshape(_N, 20),
            mu_e_p.reshape(_E, 20), lv_e_p.reshape(_E, 20))
